# baseline (speedup 1.0000x reference)
import functools

import jax
import jax.numpy as jnp
from jax import lax
from jax.experimental import pallas as pl
from jax.experimental.pallas import tpu as pltpu
from jax.experimental.pallas import tpu_sc as plsc

N_TYPES = 100000
ROW_PAD = 100_096
HALF = ROW_PAD // 2
SEQ_LEN = 200
N_HALVES = 2 * SEQ_LEN
NUM_CORES = 2
NUM_SUBCORES = 16
NW = NUM_CORES * NUM_SUBCORES
MAX_H = 13
ZBUF = 50_176
LANES = 16
OWIN = 128
SEQ_PAD = 224

_mesh = plsc.VectorSubcoreMesh(core_axis_name="c", subcore_axis_name="s")


@functools.partial(
    pl.kernel,
    out_type=jax.ShapeDtypeStruct((SEQ_LEN, 1, N_TYPES), jnp.float32),
    mesh=_mesh,
    scratch_types=[
        pltpu.VMEM((ZBUF,), jnp.float32),
        pltpu.VMEM_SHARED((HALF,), jnp.float32),
        pltpu.VMEM((MAX_H, OWIN), jnp.float32),
        pltpu.VMEM((SEQ_PAD,), jnp.int32),
        pltpu.VMEM((LANES,), jnp.float32),
        pltpu.SemaphoreType.DMA,
        pltpu.SemaphoreType.DMA,
    ],
    compiler_params=pltpu.CompilerParams(needs_layout_passes=False),
)
def _bow_sc(tokens_hbm, scale_hbm, out_hbm, zbuf, zshared, obuf, tbuf, vbuf,
            zsem, osem):
    wid = lax.axis_index("c") * NUM_SUBCORES + lax.axis_index("s")
    hstart = (wid * 25) // 2
    nh = (wid + 1) * 25 // 2 - hstart

    zv = jnp.zeros((LANES,), jnp.float32)

    def zbody(i, carry):
        b = i * (32 * LANES)
        for k in range(32):
            zbuf[pl.ds(b + k * LANES, LANES)] = zv
        return carry

    lax.fori_loop(0, ZBUF // (32 * LANES), zbody, 0)

    pl.when(lax.axis_index("s") == 0)(
        lambda: pltpu.sync_copy(zbuf.at[pl.ds(0, HALF)], zshared))
    plsc.subcore_barrier()

    rows, sides, copies = [], [], []
    for j in range(MAX_H):
        h = hstart + j
        r = h // 2
        side = h - 2 * r
        cp = pltpu.make_async_copy(
            zshared.at[pl.ds(0, HALF)],
            out_hbm.at[r, 0, pl.ds(pl.multiple_of(side * HALF, OWIN), HALF)],
            zsem,
        )
        pl.when(j < nh)(cp.start)
        rows.append(r)
        sides.append(side)
        copies.append(cp)

    pltpu.sync_copy(tokens_hbm, tbuf.at[pl.ds(0, SEQ_LEN)])
    pltpu.sync_copy(scale_hbm, vbuf.at[pl.ds(0, 1)])
    scale_v = vbuf[pl.ds(0, LANES)][0]
    offs, owns = [], []
    for j in range(MAX_H):
        t = tbuf[pl.ds(rows[j], LANES)][0]
        off = pl.multiple_of((t // OWIN) * OWIN, OWIN)
        pos = t - off
        for g in range(OWIN // LANES):
            lane = g * LANES + lax.iota(jnp.int32, LANES)
            obuf[j, pl.ds(g * LANES, LANES)] = jnp.where(
                lane == pos, scale_v, 0.0)
        offs.append(off)
        owns.append(t // HALF == sides[j])

    for j, cp in enumerate(copies):
        pl.when(j < nh)(cp.wait)
    ocopies = []
    for j in range(MAX_H):
        ocp = pltpu.make_async_copy(
            obuf.at[j],
            out_hbm.at[rows[j], 0, pl.ds(offs[j], OWIN)],
            osem,
        )
        pl.when((j < nh) & owns[j])(ocp.start)
        ocopies.append(ocp)
    for j, ocp in enumerate(ocopies):
        pl.when((j < nh) & owns[j])(ocp.wait)


def kernel(tokens, scale):
    return _bow_sc(tokens.astype(jnp.int32), scale.astype(jnp.float32))


# speedup vs baseline: 1.3515x; 1.3515x over previous
import functools

import jax
import jax.numpy as jnp
from jax import lax
from jax.experimental import pallas as pl
from jax.experimental.pallas import tpu as pltpu
from jax.experimental.pallas import tpu_sc as plsc

N_TYPES = 100000
ROW_PAD = 100_096
HALF = ROW_PAD // 2
SUBA = 195 * 128
SUBB = 196 * 128
SEQ_LEN = 200
NUM_CORES = 2
NUM_SUBCORES = 16
NW = NUM_CORES * NUM_SUBCORES
NFULL = 12
ZBUF = 50_176
LANES = 16
OWIN = 128
SEQ_PAD = 224

_mesh = plsc.VectorSubcoreMesh(core_axis_name="c", subcore_axis_name="s")


@functools.partial(
    pl.kernel,
    out_type=jax.ShapeDtypeStruct((SEQ_LEN, 1, N_TYPES), jnp.float32),
    mesh=_mesh,
    scratch_types=[
        pltpu.VMEM((ZBUF,), jnp.float32),
        pltpu.VMEM((NFULL + 1, OWIN), jnp.float32),
        pltpu.VMEM((SEQ_PAD,), jnp.int32),
        pltpu.VMEM((LANES,), jnp.float32),
        pltpu.SemaphoreType.DMA,
        pltpu.SemaphoreType.DMA,
    ],
    compiler_params=pltpu.CompilerParams(needs_layout_passes=False),
)
def _bow_sc(tokens_hbm, scale_hbm, out_hbm, zbuf, obuf, tbuf, vbuf, zsem, osem):
    wid = lax.axis_index("c") * NUM_SUBCORES + lax.axis_index("s")

    zv = jnp.zeros((LANES,), jnp.float32)

    def zbody(i, carry):
        b = i * (32 * LANES)
        for k in range(32):
            zbuf[pl.ds(b + k * LANES, LANES)] = zv
        return carry

    lax.fori_loop(0, ZBUF // (32 * LANES), zbody, 0)

    rows, sides = [], []
    for j in range(NFULL):
        rows.append(6 * wid + j // 2)
        sides.append(j % 2)
    t_row = 192 + wid // 4
    t_side = (wid // 2) % 2
    is_upper = wid % 2
    rows.append(t_row)
    sides.append(t_side)

    copies = []
    for j in range(NFULL):
        cp = pltpu.make_async_copy(
            zbuf.at[pl.ds(0, HALF)],
            out_hbm.at[rows[j], 0,
                       pl.ds(pl.multiple_of(sides[j] * HALF, OWIN), HALF)],
            zsem,
        )
        cp.start()
        copies.append(cp)
    tbase = pl.multiple_of(t_side * HALF, OWIN)
    cp_lo = pltpu.make_async_copy(
        zbuf.at[pl.ds(0, SUBA)],
        out_hbm.at[t_row, 0, pl.ds(tbase, SUBA)],
        zsem,
    )
    pl.when(is_upper == 0)(cp_lo.start)
    cp_hi = pltpu.make_async_copy(
        zbuf.at[pl.ds(0, SUBB)],
        out_hbm.at[t_row, 0, pl.ds(tbase + SUBA, SUBB)],
        zsem,
    )
    pl.when(is_upper == 1)(cp_hi.start)

    pltpu.sync_copy(tokens_hbm, tbuf.at[pl.ds(0, SEQ_LEN)])
    pltpu.sync_copy(scale_hbm, vbuf.at[pl.ds(0, 1)])
    scale_v = vbuf[pl.ds(0, LANES)][0]
    offs, owns = [], []
    for j in range(NFULL + 1):
        t = tbuf[pl.ds(rows[j], LANES)][0]
        off = pl.multiple_of((t // OWIN) * OWIN, OWIN)
        pos = t - off
        for g in range(OWIN // LANES):
            lane = g * LANES + lax.iota(jnp.int32, LANES)
            obuf[j, pl.ds(g * LANES, LANES)] = jnp.where(
                lane == pos, scale_v, 0.0)
        offs.append(off)
        own = (t // HALF) == sides[j]
        if j == NFULL:
            within = t // OWIN - t_side * (HALF // OWIN)
            own = own & ((within >= 195) == (is_upper == 1))
        owns.append(own)

    for cp in copies:
        cp.wait()
    pl.when(is_upper == 0)(cp_lo.wait)
    pl.when(is_upper == 1)(cp_hi.wait)
    ocopies = []
    for j in range(NFULL + 1):
        ocp = pltpu.make_async_copy(
            obuf.at[j],
            out_hbm.at[rows[j], 0, pl.ds(offs[j], OWIN)],
            osem,
        )
        pl.when(owns[j])(ocp.start)
        ocopies.append(ocp)
    for j, ocp in enumerate(ocopies):
        pl.when(owns[j])(ocp.wait)


def kernel(tokens, scale):
    return _bow_sc(tokens.astype(jnp.int32), scale.astype(jnp.float32))


# speedup vs baseline: 1.3560x; 1.0033x over previous
import functools

import jax
import jax.numpy as jnp
from jax import lax
from jax.experimental import pallas as pl
from jax.experimental.pallas import tpu as pltpu
from jax.experimental.pallas import tpu_sc as plsc

N_TYPES = 100000
ROW_PAD = 100_096
HALF = ROW_PAD // 2
SUBA = 195 * 128
SUBB = 196 * 128
SEQ_LEN = 200
NUM_CORES = 2
NUM_SUBCORES = 16
NW = NUM_CORES * NUM_SUBCORES
NFULL = 12
ZBUF = 25_088
LANES = 16
OWIN = 128
SEQ_PAD = 224

_mesh = plsc.VectorSubcoreMesh(core_axis_name="c", subcore_axis_name="s")


@functools.partial(
    pl.kernel,
    out_type=jax.ShapeDtypeStruct((SEQ_LEN, 1, N_TYPES), jnp.float32),
    mesh=_mesh,
    scratch_types=[
        pltpu.VMEM((ZBUF,), jnp.float32),
        pltpu.VMEM((NFULL + 1, OWIN), jnp.float32),
        pltpu.VMEM((SEQ_PAD,), jnp.int32),
        pltpu.VMEM((LANES,), jnp.float32),
        pltpu.SemaphoreType.DMA,
        pltpu.SemaphoreType.DMA,
    ],
    compiler_params=pltpu.CompilerParams(needs_layout_passes=False),
)
def _bow_sc(tokens_hbm, scale_hbm, out_hbm, zbuf, obuf, tbuf, vbuf, zsem, osem):
    wid = lax.axis_index("c") * NUM_SUBCORES + lax.axis_index("s")

    zv = jnp.zeros((LANES,), jnp.float32)

    def zbody(i, carry):
        b = i * (32 * LANES)
        for k in range(32):
            zbuf[pl.ds(b + k * LANES, LANES)] = zv
        return carry

    lax.fori_loop(0, ZBUF // (32 * LANES), zbody, 0)

    rows, sides = [], []
    for j in range(NFULL):
        rows.append(6 * wid + j // 2)
        sides.append(j % 2)
    t_row = 192 + wid // 4
    t_side = (wid // 2) % 2
    is_upper = wid % 2
    rows.append(t_row)
    sides.append(t_side)

    copies = []
    for j in range(NFULL):
        base = pl.multiple_of(sides[j] * HALF, OWIN)
        cp_a = pltpu.make_async_copy(
            zbuf.at[pl.ds(0, SUBA)],
            out_hbm.at[rows[j], 0, pl.ds(base, SUBA)],
            zsem,
        )
        cp_a.start()
        copies.append(cp_a)
        cp_b = pltpu.make_async_copy(
            zbuf.at[pl.ds(0, SUBB)],
            out_hbm.at[rows[j], 0, pl.ds(base + SUBA, SUBB)],
            zsem,
        )
        cp_b.start()
        copies.append(cp_b)
    tbase = pl.multiple_of(t_side * HALF, OWIN)
    cp_lo = pltpu.make_async_copy(
        zbuf.at[pl.ds(0, SUBA)],
        out_hbm.at[t_row, 0, pl.ds(tbase, SUBA)],
        zsem,
    )
    pl.when(is_upper == 0)(cp_lo.start)
    cp_hi = pltpu.make_async_copy(
        zbuf.at[pl.ds(0, SUBB)],
        out_hbm.at[t_row, 0, pl.ds(tbase + SUBA, SUBB)],
        zsem,
    )
    pl.when(is_upper == 1)(cp_hi.start)

    pltpu.sync_copy(tokens_hbm, tbuf.at[pl.ds(0, SEQ_LEN)])
    pltpu.sync_copy(scale_hbm, vbuf.at[pl.ds(0, 1)])
    scale_v = vbuf[pl.ds(0, LANES)][0]
    offs, owns = [], []
    for j in range(NFULL + 1):
        t = tbuf[pl.ds(rows[j], LANES)][0]
        off = pl.multiple_of((t // OWIN) * OWIN, OWIN)
        pos = t - off
        for g in range(OWIN // LANES):
            lane = g * LANES + lax.iota(jnp.int32, LANES)
            obuf[j, pl.ds(g * LANES, LANES)] = jnp.where(
                lane == pos, scale_v, 0.0)
        offs.append(off)
        own = (t // HALF) == sides[j]
        if j == NFULL:
            within = t // OWIN - t_side * (HALF // OWIN)
            own = own & ((within >= 195) == (is_upper == 1))
        owns.append(own)

    for cp in copies:
        cp.wait()
    pl.when(is_upper == 0)(cp_lo.wait)
    pl.when(is_upper == 1)(cp_hi.wait)
    ocopies = []
    for j in range(NFULL + 1):
        ocp = pltpu.make_async_copy(
            obuf.at[j],
            out_hbm.at[rows[j], 0, pl.ds(offs[j], OWIN)],
            osem,
        )
        pl.when(owns[j])(ocp.start)
        ocopies.append(ocp)
    for j, ocp in enumerate(ocopies):
        pl.when(owns[j])(ocp.wait)


def kernel(tokens, scale):
    return _bow_sc(tokens.astype(jnp.int32), scale.astype(jnp.float32))
